# baseline (speedup 1.0000x reference)
import jax
import jax.numpy as jnp
from jax import lax
from jax.experimental import pallas as pl
from jax.experimental.pallas import tpu as pltpu

_NCOPY = 8


def _writer_kernel(slot_ref, hidden_ref, wk_ref, bk_ref, wv_ref, bv_ref,
                   wg_ref, bg_ref, keys_ref, vals_ref,
                   out_keys_ref, out_vals_ref, gate_ref,
                   cur_k, cur_v, copy_sem, row_sem):
    B = keys_ref.shape[0]
    D = hidden_ref.shape[1]
    off = slot_ref[0] * D

    cb = B // _NCOPY
    copies = []
    for a, (src, dst) in enumerate(((keys_ref, out_keys_ref),
                                    (vals_ref, out_vals_ref))):
        for c in range(_NCOPY):
            cp = pltpu.make_async_copy(src.at[pl.ds(c * cb, cb)],
                                       dst.at[pl.ds(c * cb, cb)],
                                       copy_sem.at[a, c])
            cp.start()
            copies.append(cp)

    gk = pltpu.make_async_copy(keys_ref.at[:, pl.ds(off, D)], cur_k,
                               row_sem.at[0])
    gv = pltpu.make_async_copy(vals_ref.at[:, pl.ds(off, D)], cur_v,
                               row_sem.at[1])
    gk.start()
    gv.start()

    h = hidden_ref[...]
    dn = (((1,), (1,)), ((), ()))
    nk = jnp.tanh(lax.dot_general(h, wk_ref[...], dn,
                                  preferred_element_type=jnp.float32)
                  + bk_ref[...])
    nv = jnp.tanh(lax.dot_general(h, wv_ref[...], dn,
                                  preferred_element_type=jnp.float32)
                  + bv_ref[...])
    g = jax.nn.sigmoid(lax.dot_general(h, wg_ref[...], dn,
                                       preferred_element_type=jnp.float32)
                       + bg_ref[...])

    gk.wait()
    gv.wait()
    cur_k[...] = cur_k[...] * (1.0 - g) + nk * g
    cur_v[...] = cur_v[...] * (1.0 - g) + nv * g
    gate_ref[...] = g[:, :gate_ref.shape[1]]

    for cp in copies:
        cp.wait()

    sk = pltpu.make_async_copy(cur_k, out_keys_ref.at[:, pl.ds(off, D)],
                               row_sem.at[0])
    sv = pltpu.make_async_copy(cur_v, out_vals_ref.at[:, pl.ds(off, D)],
                               row_sem.at[1])
    sk.start()
    sv.start()
    sk.wait()
    sv.wait()


def kernel(hidden, working_keys, working_values, step, Wk, bk, Wv, bv, Wg, bg):
    B, S, D = working_keys.shape
    slot = (jnp.asarray(step, jnp.int32) % S).reshape(1)
    k2 = working_keys.reshape(B, S * D)
    v2 = working_values.reshape(B, S * D)

    hbm = pl.BlockSpec(memory_space=pltpu.MemorySpace.HBM)
    vmem = pl.BlockSpec(memory_space=pltpu.MemorySpace.VMEM)
    smem = pl.BlockSpec(memory_space=pltpu.MemorySpace.SMEM)

    out_keys, out_vals, gate = pl.pallas_call(
        _writer_kernel,
        out_shape=[
            jax.ShapeDtypeStruct((B, S * D), jnp.float32),
            jax.ShapeDtypeStruct((B, S * D), jnp.float32),
            jax.ShapeDtypeStruct((B, 128), jnp.float32),
        ],
        in_specs=[smem, vmem, vmem, vmem, vmem, vmem, vmem, vmem, hbm, hbm],
        out_specs=[hbm, hbm, vmem],
        scratch_shapes=[
            pltpu.VMEM((B, D), jnp.float32),
            pltpu.VMEM((B, D), jnp.float32),
            pltpu.SemaphoreType.DMA((2, _NCOPY)),
            pltpu.SemaphoreType.DMA((2,)),
        ],
    )(slot, hidden, Wk, bk.reshape(1, D), Wv, bv.reshape(1, D),
      jnp.broadcast_to(Wg, (D, D)), jnp.broadcast_to(bg.reshape(1, 1), (1, D)),
      k2, v2)

    return (out_keys.reshape(B, S, D), out_vals.reshape(B, S, D), gate[:, 0])


# speedup vs baseline: 49.8852x; 49.8852x over previous
import jax
import jax.numpy as jnp
from jax import lax
from jax.experimental import pallas as pl
from jax.experimental.pallas import tpu as pltpu

_BB = 64


def _writer_kernel(slot_ref, hidden_ref, wk_ref, bk_ref, wv_ref, bv_ref,
                   wg_ref, bg_ref, keys_ref, vals_ref,
                   out_keys_ref, out_vals_ref, gate_ref):
    S = keys_ref.shape[1]
    slot = slot_ref[0]

    h = hidden_ref[...]
    dn = (((1,), (1,)), ((), ()))
    nk = jnp.tanh(lax.dot_general(h, wk_ref[...], dn,
                                  preferred_element_type=jnp.float32)
                  + bk_ref[...])
    nv = jnp.tanh(lax.dot_general(h, wv_ref[...], dn,
                                  preferred_element_type=jnp.float32)
                  + bv_ref[...])
    g = jax.nn.sigmoid(lax.dot_general(h, wg_ref[...], dn,
                                       preferred_element_type=jnp.float32)
                       + bg_ref[...])

    in_k = keys_ref[...]
    in_v = vals_ref[...]
    cur_k = keys_ref[:, slot, :]
    cur_v = vals_ref[:, slot, :]
    blend_k = cur_k * (1.0 - g) + nk * g
    blend_v = cur_v * (1.0 - g) + nv * g

    sel = lax.broadcasted_iota(jnp.int32, (1, S, 1), 1) == slot
    out_keys_ref[...] = jnp.where(sel, blend_k[:, None, :], in_k)
    out_vals_ref[...] = jnp.where(sel, blend_v[:, None, :], in_v)
    gate_ref[...] = g[:, :gate_ref.shape[1]]


def kernel(hidden, working_keys, working_values, step, Wk, bk, Wv, bv, Wg, bg):
    B, S, D = working_keys.shape
    slot = (jnp.asarray(step, jnp.int32) % S).reshape(1)

    smem = pl.BlockSpec(memory_space=pltpu.MemorySpace.SMEM)
    full = lambda shape: pl.BlockSpec(shape, lambda i: (0,) * len(shape))
    bblk = pl.BlockSpec((_BB, S, D), lambda i: (i, 0, 0))

    out_keys, out_vals, gate = pl.pallas_call(
        _writer_kernel,
        grid=(B // _BB,),
        out_shape=[
            jax.ShapeDtypeStruct((B, S, D), jnp.float32),
            jax.ShapeDtypeStruct((B, S, D), jnp.float32),
            jax.ShapeDtypeStruct((B, 128), jnp.float32),
        ],
        in_specs=[
            smem,
            pl.BlockSpec((_BB, D), lambda i: (i, 0)),
            full((D, D)), full((1, D)), full((D, D)), full((1, D)),
            full((D, D)), full((1, D)),
            bblk, bblk,
        ],
        out_specs=[bblk, bblk, pl.BlockSpec((_BB, 128), lambda i: (i, 0))],
    )(slot, hidden, Wk, bk.reshape(1, D), Wv, bv.reshape(1, D),
      jnp.broadcast_to(Wg, (D, D)), jnp.broadcast_to(bg.reshape(1, 1), (1, D)),
      working_keys, working_values)

    return (out_keys, out_vals, gate[:, 0])
